# Initial kernel scaffold; baseline (speedup 1.0000x reference)
import jax
import jax.numpy as jnp
from jax.experimental import pallas as pl


def kernel(user_emb, item_emb, edge_index):
    raise NotImplementedError("write your pallas kernel here")


# speedup vs baseline: 10.9472x; 10.9472x over previous
import functools

import jax
import jax.numpy as jnp
from jax import lax
from jax.experimental import pallas as pl
from jax.experimental.pallas import tpu as pltpu
from jax.experimental.pallas import tpu_sc as plsc

NU = 25000
D = 64
E = 400000
TILES = 16
LANES = 16
CHUNK = 128
PAD_HALF = 25088
NFULL = 2 * PAD_HALF
ROWS_PT = PAD_HALF // TILES
EPAD = 401408
NCH = EPAD // (TILES * CHUNK)
PAD_ROW = NU

_mesh = plsc.VectorSubcoreMesh(core_axis_name="c", subcore_axis_name="s")


def _zero_buf(rows_v):
    def body(r, _):
        for j in range(D // LANES):
            rows_v[r, pl.ds(j * LANES, LANES)] = jnp.zeros((LANES,), jnp.float32)
        return 0
    lax.fori_loop(0, CHUNK, body, 0)


@functools.partial(
    pl.kernel,
    out_type=jax.ShapeDtypeStruct((NFULL,), jnp.float32),
    mesh=_mesh,
    scratch_types=[
        pltpu.VMEM((CHUNK,), jnp.int32),
        pltpu.VMEM((CHUNK,), jnp.float32),
        pltpu.VMEM((ROWS_PT,), jnp.float32),
        pltpu.VMEM_SHARED((PAD_HALF,), jnp.float32),
    ],
)
def _sc_degree(dst_hbm, deg_hbm, idx_v, ones_v, zrow_v, deg_sh):
    c = lax.axis_index("c")
    s = lax.axis_index("s")

    def fill(r, _):
        ones_v[pl.ds(r * LANES, LANES)] = jnp.ones((LANES,), jnp.float32)
        return 0
    lax.fori_loop(0, CHUNK // LANES, fill, 0)

    def zfill(r, _):
        zrow_v[pl.ds(r * LANES, LANES)] = jnp.zeros((LANES,), jnp.float32)
        return 0
    lax.fori_loop(0, ROWS_PT // LANES, zfill, 0)
    pltpu.sync_copy(zrow_v, deg_sh.at[pl.ds(s * ROWS_PT, ROWS_PT)])
    plsc.subcore_barrier()

    def edge(k, _):
        pltpu.sync_copy(dst_hbm.at[c, s, k], idx_v)
        pltpu.sync_copy(ones_v, deg_sh.at[idx_v], add=True)
        return 0
    lax.fori_loop(0, NCH, edge, 0)
    plsc.subcore_barrier()

    pltpu.sync_copy(deg_sh.at[pl.ds(s * ROWS_PT, ROWS_PT)], zrow_v)
    pltpu.sync_copy(zrow_v, deg_hbm.at[pl.ds(c * PAD_HALF + s * ROWS_PT, ROWS_PT)])


@functools.partial(
    pl.kernel,
    out_type=jax.ShapeDtypeStruct((NFULL, D), jnp.float32),
    mesh=_mesh,
    scratch_types=[
        pltpu.VMEM((CHUNK,), jnp.int32),
        pltpu.VMEM((CHUNK,), jnp.int32),
        pltpu.VMEM((CHUNK, D), jnp.float32),
        pltpu.VMEM_SHARED((PAD_HALF, D), jnp.float32),
        pltpu.SemaphoreType.DMA,
    ],
    compiler_params=pltpu.CompilerParams(use_tc_tiling_on_sc=False),
)
def _sc_spmm(src_hbm, dst_hbm, y_hbm, out_hbm, sidx_v, didx_v, rows_v, acc_sh, sem):
    c = lax.axis_index("c")
    s = lax.axis_index("s")
    base = s * ROWS_PT

    _zero_buf(rows_v)
    def zcp(k, _):
        pltpu.sync_copy(rows_v, acc_sh.at[pl.ds(base + k * CHUNK, CHUNK)])
        return 0
    lax.fori_loop(0, ROWS_PT // CHUNK, zcp, 0)
    rem = ROWS_PT - (ROWS_PT // CHUNK) * CHUNK
    if rem:
        pltpu.sync_copy(
            rows_v.at[pl.ds(0, rem)],
            acc_sh.at[pl.ds(base + (ROWS_PT // CHUNK) * CHUNK, rem)],
        )
    plsc.subcore_barrier()

    def edge(k, _):
        pltpu.sync_copy(src_hbm.at[c, s, k], sidx_v)
        pltpu.sync_copy(dst_hbm.at[c, s, k], didx_v)
        pltpu.async_copy(y_hbm.at[sidx_v], rows_v, sem).wait()
        pltpu.sync_copy(rows_v, acc_sh.at[didx_v], add=True)
        return 0
    lax.fori_loop(0, NCH, edge, 0)
    plsc.subcore_barrier()

    gbase = c * PAD_HALF + base
    def wout(k, _):
        pltpu.sync_copy(acc_sh.at[pl.ds(base + k * CHUNK, CHUNK)], rows_v)
        pltpu.sync_copy(rows_v, out_hbm.at[pl.ds(gbase + k * CHUNK, CHUNK)])
        return 0
    lax.fori_loop(0, ROWS_PT // CHUNK, wout, 0)
    if rem:
        tail = (ROWS_PT // CHUNK) * CHUNK
        pltpu.sync_copy(acc_sh.at[pl.ds(base + tail, rem)], rows_v.at[pl.ds(0, rem)])
        pltpu.sync_copy(rows_v.at[pl.ds(0, rem)], out_hbm.at[pl.ds(gbase + tail, rem)])


_BLK = 448


def _tc_scale0_body(deg_ref, x_ref, dinv_ref, y_ref):
    d = deg_ref[...]
    dinv = jnp.where(d > 0, lax.rsqrt(jnp.where(d > 0, d, 1.0)), 0.0)
    dinv_ref[...] = dinv
    y_ref[...] = dinv * x_ref[...]


def _tc_scale_mid_body(acc_ref, dinv_ref, sp_ref, y_ref, s_ref):
    dinv = dinv_ref[...]
    x = dinv * acc_ref[...]
    y_ref[...] = dinv * x
    s_ref[...] = sp_ref[...] + x


def _tc_scale_fin_body(acc_ref, dinv_ref, sp_ref, s_ref):
    x = dinv_ref[...] * acc_ref[...]
    s_ref[...] = (sp_ref[...] + x) * 0.25


def _rows(block_cols):
    return pl.BlockSpec((_BLK, block_cols), lambda i: (i, 0))


def _tc_scale0(deg2, x0):
    return pl.pallas_call(
        _tc_scale0_body,
        grid=(NFULL // _BLK,),
        in_specs=[_rows(1), _rows(D)],
        out_specs=[_rows(1), _rows(D)],
        out_shape=[
            jax.ShapeDtypeStruct((NFULL, 1), jnp.float32),
            jax.ShapeDtypeStruct((NFULL, D), jnp.float32),
        ],
    )(deg2, x0)


def _tc_scale_mid(acc, dinv, sp):
    return pl.pallas_call(
        _tc_scale_mid_body,
        grid=(NFULL // _BLK,),
        in_specs=[_rows(D), _rows(1), _rows(D)],
        out_specs=[_rows(D), _rows(D)],
        out_shape=[
            jax.ShapeDtypeStruct((NFULL, D), jnp.float32),
            jax.ShapeDtypeStruct((NFULL, D), jnp.float32),
        ],
    )(acc, dinv, sp)


def _tc_scale_fin(acc, dinv, sp):
    return pl.pallas_call(
        _tc_scale_fin_body,
        grid=(NFULL // _BLK,),
        in_specs=[_rows(D), _rows(1), _rows(D)],
        out_specs=_rows(D),
        out_shape=jax.ShapeDtypeStruct((NFULL, D), jnp.float32),
    )(acc, dinv, sp)


def _pad_edges(idx, pad_val):
    out = jnp.full((EPAD,), pad_val, jnp.int32)
    out = lax.dynamic_update_slice(out, idx.astype(jnp.int32), (0,))
    return out.reshape(TILES, NCH, CHUNK)


@jax.jit
def kernel(user_emb, item_emb, edge_index):
    u = edge_index[0]
    i = edge_index[1]

    src = jnp.stack([_pad_edges(i + PAD_HALF, PAD_ROW),
                     _pad_edges(u, PAD_ROW)])
    dst = jnp.stack([_pad_edges(u, PAD_ROW),
                     _pad_edges(i, PAD_ROW)])

    x0 = jnp.zeros((NFULL, D), jnp.float32)
    x0 = lax.dynamic_update_slice(x0, user_emb, (0, 0))
    x0 = lax.dynamic_update_slice(x0, item_emb, (PAD_HALF, 0))

    deg = _sc_degree(dst)
    dinv, y = _tc_scale0(deg.reshape(NFULL, 1), x0)

    s = x0
    for layer in range(3):
        acc = _sc_spmm(src, dst, y)
        if layer < 2:
            y, s = _tc_scale_mid(acc, dinv, s)
        else:
            s = _tc_scale_fin(acc, dinv, s)

    return s[:NU], s[PAD_HALF:PAD_HALF + NU]


# speedup vs baseline: 16.3281x; 1.4915x over previous
import functools

import jax
import jax.numpy as jnp
from jax import lax
from jax.experimental import pallas as pl
from jax.experimental.pallas import tpu as pltpu
from jax.experimental.pallas import tpu_sc as plsc

NU = 25000
D = 64
E = 400000
TILES = 16
LANES = 16
CHUNK = 128
PAD_HALF = 25088
NFULL = 2 * PAD_HALF
ROWS_PT = PAD_HALF // TILES
EPAD = 401408
NCH = EPAD // (TILES * CHUNK)
PAD_ROW = NU

_mesh = plsc.VectorSubcoreMesh(core_axis_name="c", subcore_axis_name="s")


def _zero_buf(rows_v):
    def body(r, _):
        for j in range(D // LANES):
            rows_v[r, pl.ds(j * LANES, LANES)] = jnp.zeros((LANES,), jnp.float32)
        return 0
    lax.fori_loop(0, CHUNK, body, 0)


@functools.partial(
    pl.kernel,
    out_type=jax.ShapeDtypeStruct((NFULL,), jnp.float32),
    mesh=_mesh,
    scratch_types=[
        pltpu.VMEM((CHUNK,), jnp.int32),
        pltpu.VMEM((CHUNK,), jnp.float32),
        pltpu.VMEM((ROWS_PT,), jnp.float32),
        pltpu.VMEM_SHARED((PAD_HALF,), jnp.float32),
    ],
)
def _sc_degree(dst_hbm, deg_hbm, idx_v, ones_v, zrow_v, deg_sh):
    c = lax.axis_index("c")
    s = lax.axis_index("s")

    def fill(r, _):
        ones_v[pl.ds(r * LANES, LANES)] = jnp.ones((LANES,), jnp.float32)
        return 0
    lax.fori_loop(0, CHUNK // LANES, fill, 0)

    def zfill(r, _):
        zrow_v[pl.ds(r * LANES, LANES)] = jnp.zeros((LANES,), jnp.float32)
        return 0
    lax.fori_loop(0, ROWS_PT // LANES, zfill, 0)
    pltpu.sync_copy(zrow_v, deg_sh.at[pl.ds(s * ROWS_PT, ROWS_PT)])
    plsc.subcore_barrier()

    def edge(k, _):
        pltpu.sync_copy(dst_hbm.at[c, s, k, 1], idx_v)
        pltpu.sync_copy(ones_v, deg_sh.at[idx_v], add=True)
        return 0
    lax.fori_loop(0, NCH, edge, 0)
    plsc.subcore_barrier()

    pltpu.sync_copy(deg_sh.at[pl.ds(s * ROWS_PT, ROWS_PT)], zrow_v)
    pltpu.sync_copy(zrow_v, deg_hbm.at[pl.ds(c * PAD_HALF + s * ROWS_PT, ROWS_PT)])


@functools.partial(
    pl.kernel,
    out_type=jax.ShapeDtypeStruct((NFULL, D), jnp.float32),
    mesh=_mesh,
    scratch_types=[
        pltpu.VMEM((2, CHUNK), jnp.int32),
        pltpu.VMEM((2, CHUNK), jnp.int32),
        pltpu.VMEM((CHUNK, D), jnp.float32),
        pltpu.VMEM((CHUNK, D), jnp.float32),
        pltpu.VMEM_SHARED((PAD_HALF, D), jnp.float32),
        pltpu.SemaphoreType.DMA,
        pltpu.SemaphoreType.DMA,
    ],
    compiler_params=pltpu.CompilerParams(use_tc_tiling_on_sc=False),
)
def _sc_spmm(ed_hbm, y_hbm, out_hbm, ida, idb, rows_a, rows_b, acc_sh, sem_a, sem_b):
    c = lax.axis_index("c")
    s = lax.axis_index("s")
    base = s * ROWS_PT

    _zero_buf(rows_a)
    def zcp(k, _):
        pltpu.sync_copy(rows_a, acc_sh.at[pl.ds(base + k * CHUNK, CHUNK)])
        return 0
    lax.fori_loop(0, ROWS_PT // CHUNK, zcp, 0)
    rem = ROWS_PT - (ROWS_PT // CHUNK) * CHUNK
    if rem:
        pltpu.sync_copy(
            rows_a.at[pl.ds(0, rem)],
            acc_sh.at[pl.ds(base + (ROWS_PT // CHUNK) * CHUNK, rem)],
        )
    plsc.subcore_barrier()

    pltpu.sync_copy(ed_hbm.at[c, s, 0], ida)
    pltpu.async_copy(y_hbm.at[ida.at[0]], rows_a, sem_a)
    pltpu.sync_copy(ed_hbm.at[c, s, 1], idb)

    def edge(j, _):
        pltpu.async_copy(y_hbm.at[idb.at[0]], rows_b, sem_b)
        pltpu.make_async_copy(y_hbm.at[ida.at[0]], rows_a, sem_a).wait()
        pltpu.sync_copy(rows_a, acc_sh.at[ida.at[1]], add=True)

        @pl.when(j < NCH // 2 - 1)
        def _():
            pltpu.sync_copy(ed_hbm.at[c, s, 2 * j + 2], ida)
            pltpu.async_copy(y_hbm.at[ida.at[0]], rows_a, sem_a)

        pltpu.make_async_copy(y_hbm.at[idb.at[0]], rows_b, sem_b).wait()
        pltpu.sync_copy(rows_b, acc_sh.at[idb.at[1]], add=True)

        @pl.when(j < NCH // 2 - 1)
        def _():
            pltpu.sync_copy(ed_hbm.at[c, s, 2 * j + 3], idb)
        return 0
    lax.fori_loop(0, NCH // 2, edge, 0)
    plsc.subcore_barrier()

    gbase = c * PAD_HALF + base
    def wout(k, _):
        pltpu.sync_copy(acc_sh.at[pl.ds(base + k * CHUNK, CHUNK)], rows_a)
        pltpu.sync_copy(rows_a, out_hbm.at[pl.ds(gbase + k * CHUNK, CHUNK)])
        return 0
    lax.fori_loop(0, ROWS_PT // CHUNK, wout, 0)
    if rem:
        tail = (ROWS_PT // CHUNK) * CHUNK
        pltpu.sync_copy(acc_sh.at[pl.ds(base + tail, rem)], rows_a.at[pl.ds(0, rem)])
        pltpu.sync_copy(rows_a.at[pl.ds(0, rem)], out_hbm.at[pl.ds(gbase + tail, rem)])


_BLK = 448


def _tc_scale0_body(deg_ref, x_ref, dinv_ref, y_ref):
    d = deg_ref[...]
    dinv = jnp.where(d > 0, lax.rsqrt(jnp.where(d > 0, d, 1.0)), 0.0)
    dinv_ref[...] = dinv
    y_ref[...] = dinv * x_ref[...]


def _tc_scale_mid_body(acc_ref, dinv_ref, sp_ref, y_ref, s_ref):
    dinv = dinv_ref[...]
    x = dinv * acc_ref[...]
    y_ref[...] = dinv * x
    s_ref[...] = sp_ref[...] + x


def _tc_scale_fin_body(acc_ref, dinv_ref, sp_ref, s_ref):
    x = dinv_ref[...] * acc_ref[...]
    s_ref[...] = (sp_ref[...] + x) * 0.25


def _rows(block_cols):
    return pl.BlockSpec((_BLK, block_cols), lambda i: (i, 0))


def _tc_scale0(deg2, x0):
    return pl.pallas_call(
        _tc_scale0_body,
        grid=(NFULL // _BLK,),
        in_specs=[_rows(1), _rows(D)],
        out_specs=[_rows(1), _rows(D)],
        out_shape=[
            jax.ShapeDtypeStruct((NFULL, 1), jnp.float32),
            jax.ShapeDtypeStruct((NFULL, D), jnp.float32),
        ],
    )(deg2, x0)


def _tc_scale_mid(acc, dinv, sp):
    return pl.pallas_call(
        _tc_scale_mid_body,
        grid=(NFULL // _BLK,),
        in_specs=[_rows(D), _rows(1), _rows(D)],
        out_specs=[_rows(D), _rows(D)],
        out_shape=[
            jax.ShapeDtypeStruct((NFULL, D), jnp.float32),
            jax.ShapeDtypeStruct((NFULL, D), jnp.float32),
        ],
    )(acc, dinv, sp)


def _tc_scale_fin(acc, dinv, sp):
    return pl.pallas_call(
        _tc_scale_fin_body,
        grid=(NFULL // _BLK,),
        in_specs=[_rows(D), _rows(1), _rows(D)],
        out_specs=_rows(D),
        out_shape=jax.ShapeDtypeStruct((NFULL, D), jnp.float32),
    )(acc, dinv, sp)


def _pad_edges(idx, pad_val):
    out = jnp.full((EPAD,), pad_val, jnp.int32)
    out = lax.dynamic_update_slice(out, idx.astype(jnp.int32), (0,))
    return out.reshape(TILES, NCH, CHUNK)


@jax.jit
def kernel(user_emb, item_emb, edge_index):
    u = edge_index[0]
    i = edge_index[1]

    ed = jnp.stack([
        jnp.stack([_pad_edges(i + PAD_HALF, PAD_ROW), _pad_edges(u, PAD_ROW)], axis=2),
        jnp.stack([_pad_edges(u, PAD_ROW), _pad_edges(i, PAD_ROW)], axis=2),
    ])

    x0 = jnp.zeros((NFULL, D), jnp.float32)
    x0 = lax.dynamic_update_slice(x0, user_emb, (0, 0))
    x0 = lax.dynamic_update_slice(x0, item_emb, (PAD_HALF, 0))

    deg = _sc_degree(ed)
    dinv, y = _tc_scale0(deg.reshape(NFULL, 1), x0)

    s = x0
    for layer in range(3):
        acc = _sc_spmm(ed, y)
        if layer < 2:
            y, s = _tc_scale_mid(acc, dinv, s)
        else:
            s = _tc_scale_fin(acc, dinv, s)

    return s[:NU], s[PAD_HALF:PAD_HALF + NU]


# speedup vs baseline: 18.3810x; 1.1257x over previous
import functools

import jax
import jax.numpy as jnp
from jax import lax
from jax.experimental import pallas as pl
from jax.experimental.pallas import tpu as pltpu
from jax.experimental.pallas import tpu_sc as plsc

NU = 25000
D = 64
E = 400000
TILES = 16
LANES = 16
CHUNK = 128
PAD_HALF = 25088
NFULL = 2 * PAD_HALF
ROWS_PT = PAD_HALF // TILES
EPAD = 401408
NCH = EPAD // (TILES * CHUNK)
PAD_ROW = NU

_mesh = plsc.VectorSubcoreMesh(core_axis_name="c", subcore_axis_name="s")


def _zero_buf(rows_v):
    def body(r, _):
        for j in range(D // LANES):
            rows_v[r, pl.ds(j * LANES, LANES)] = jnp.zeros((LANES,), jnp.float32)
        return 0
    lax.fori_loop(0, CHUNK, body, 0)


@functools.partial(
    pl.kernel,
    out_type=jax.ShapeDtypeStruct((NFULL,), jnp.float32),
    mesh=_mesh,
    scratch_types=[
        pltpu.VMEM((CHUNK,), jnp.int32),
        pltpu.VMEM((CHUNK,), jnp.float32),
        pltpu.VMEM((ROWS_PT,), jnp.float32),
        pltpu.VMEM_SHARED((PAD_HALF,), jnp.float32),
    ],
)
def _sc_degree(dst_hbm, deg_hbm, idx_v, ones_v, zrow_v, deg_sh):
    c = lax.axis_index("c")
    s = lax.axis_index("s")

    def fill(r, _):
        ones_v[pl.ds(r * LANES, LANES)] = jnp.ones((LANES,), jnp.float32)
        return 0
    lax.fori_loop(0, CHUNK // LANES, fill, 0)

    def zfill(r, _):
        zrow_v[pl.ds(r * LANES, LANES)] = jnp.zeros((LANES,), jnp.float32)
        return 0
    lax.fori_loop(0, ROWS_PT // LANES, zfill, 0)
    pltpu.sync_copy(zrow_v, deg_sh.at[pl.ds(s * ROWS_PT, ROWS_PT)])
    plsc.subcore_barrier()

    def edge(k, _):
        pltpu.sync_copy(dst_hbm.at[c, s, k, 1], idx_v)
        pltpu.sync_copy(ones_v, deg_sh.at[idx_v], add=True)
        return 0
    lax.fori_loop(0, NCH, edge, 0)
    plsc.subcore_barrier()

    pltpu.sync_copy(deg_sh.at[pl.ds(s * ROWS_PT, ROWS_PT)], zrow_v)
    pltpu.sync_copy(zrow_v, deg_hbm.at[pl.ds(c * PAD_HALF + s * ROWS_PT, ROWS_PT)])


@functools.partial(
    pl.kernel,
    out_type=jax.ShapeDtypeStruct((NFULL, D), jnp.float32),
    mesh=_mesh,
    scratch_types=[
        pltpu.VMEM((2, CHUNK), jnp.int32),
        pltpu.VMEM((2, CHUNK), jnp.int32),
        pltpu.VMEM((CHUNK, D), jnp.float32),
        pltpu.VMEM((CHUNK, D), jnp.float32),
        pltpu.VMEM((CHUNK,), jnp.float32),
        pltpu.VMEM_SHARED((PAD_HALF, D), jnp.float32),
        pltpu.SemaphoreType.DMA,
        pltpu.SemaphoreType.DMA,
    ],
    compiler_params=pltpu.CompilerParams(
        use_tc_tiling_on_sc=False, needs_layout_passes=False),
)
def _sc_spmm(ed_hbm, y_hbm, dinv2_hbm, out_hbm, ida, idb, rows_a, rows_b, dv_v,
             acc_sh, sem_a, sem_b):
    c = lax.axis_index("c")
    s = lax.axis_index("s")
    base = s * ROWS_PT

    _zero_buf(rows_a)
    def zcp(k, _):
        pltpu.sync_copy(rows_a, acc_sh.at[pl.ds(base + k * CHUNK, CHUNK)])
        return 0
    lax.fori_loop(0, ROWS_PT // CHUNK, zcp, 0)
    rem = ROWS_PT - (ROWS_PT // CHUNK) * CHUNK
    if rem:
        pltpu.sync_copy(
            rows_a.at[pl.ds(0, rem)],
            acc_sh.at[pl.ds(base + (ROWS_PT // CHUNK) * CHUNK, rem)],
        )
    plsc.subcore_barrier()

    pltpu.sync_copy(ed_hbm.at[c, s, 0], ida)
    pltpu.async_copy(y_hbm.at[ida.at[0]], rows_a, sem_a)
    pltpu.sync_copy(ed_hbm.at[c, s, 1], idb)

    def edge(j, _):
        pltpu.async_copy(y_hbm.at[idb.at[0]], rows_b, sem_b)
        pltpu.make_async_copy(y_hbm.at[ida.at[0]], rows_a, sem_a).wait()
        pltpu.sync_copy(rows_a, acc_sh.at[ida.at[1]], add=True)

        @pl.when(j < NCH // 2 - 1)
        def _():
            pltpu.sync_copy(ed_hbm.at[c, s, 2 * j + 2], ida)
            pltpu.async_copy(y_hbm.at[ida.at[0]], rows_a, sem_a)

        pltpu.make_async_copy(y_hbm.at[idb.at[0]], rows_b, sem_b).wait()
        pltpu.sync_copy(rows_b, acc_sh.at[idb.at[1]], add=True)

        @pl.when(j < NCH // 2 - 1)
        def _():
            pltpu.sync_copy(ed_hbm.at[c, s, 2 * j + 3], idb)
        return 0
    lax.fori_loop(0, NCH // 2, edge, 0)
    plsc.subcore_barrier()

    gbase = c * PAD_HALF + base

    def scale_rows(nrows):
        def srow(r, _):
            dv = plsc.load_gather(dv_v, [jnp.full((LANES,), r, jnp.int32)])
            for j in range(D // LANES):
                sl = pl.ds(j * LANES, LANES)
                rows_a[r, sl] = dv * rows_a[r, sl]
            return 0
        lax.fori_loop(0, nrows, srow, 0)

    def wout(k, _):
        pltpu.sync_copy(acc_sh.at[pl.ds(base + k * CHUNK, CHUNK)], rows_a)
        pltpu.sync_copy(dinv2_hbm.at[pl.ds(gbase + k * CHUNK, CHUNK)], dv_v)
        scale_rows(CHUNK)
        pltpu.sync_copy(rows_a, out_hbm.at[pl.ds(gbase + k * CHUNK, CHUNK)])
        return 0
    lax.fori_loop(0, ROWS_PT // CHUNK, wout, 0)
    if rem:
        tail = (ROWS_PT // CHUNK) * CHUNK
        pltpu.sync_copy(acc_sh.at[pl.ds(base + tail, rem)], rows_a.at[pl.ds(0, rem)])
        pltpu.sync_copy(dinv2_hbm.at[pl.ds(gbase + tail, rem)], dv_v.at[pl.ds(0, rem)])
        scale_rows(rem)
        pltpu.sync_copy(rows_a.at[pl.ds(0, rem)], out_hbm.at[pl.ds(gbase + tail, rem)])


_BLK = 448


def _tc_scale0_body(deg_ref, x_ref, y_ref, dinv2_ref, sqdeg_ref):
    d = deg_ref[...]
    pos = d > 0
    dsafe = jnp.where(pos, d, 1.0)
    dinv = jnp.where(pos, lax.rsqrt(dsafe), 0.0)
    dinv2_ref[...] = jnp.where(pos, 1.0 / dsafe, 0.0)
    sqdeg_ref[...] = jnp.sqrt(d)
    y_ref[...] = dinv * x_ref[...]


def _tc_final_body(x0_ref, sq_ref, y1_ref, y2_ref, y3_ref, s_ref):
    ysum = y1_ref[...] + y2_ref[...] + y3_ref[...]
    s_ref[...] = (x0_ref[...] + sq_ref[...] * ysum) * 0.25


def _rows(block_cols):
    return pl.BlockSpec((_BLK, block_cols), lambda i: (i, 0))


def _tc_scale0(deg2, x0):
    return pl.pallas_call(
        _tc_scale0_body,
        grid=(NFULL // _BLK,),
        in_specs=[_rows(1), _rows(D)],
        out_specs=[_rows(D), _rows(1), _rows(1)],
        out_shape=[
            jax.ShapeDtypeStruct((NFULL, D), jnp.float32),
            jax.ShapeDtypeStruct((NFULL, 1), jnp.float32),
            jax.ShapeDtypeStruct((NFULL, 1), jnp.float32),
        ],
    )(deg2, x0)


def _tc_final(x0, sqdeg, y1, y2, y3):
    return pl.pallas_call(
        _tc_final_body,
        grid=(NFULL // _BLK,),
        in_specs=[_rows(D), _rows(1), _rows(D), _rows(D), _rows(D)],
        out_specs=_rows(D),
        out_shape=jax.ShapeDtypeStruct((NFULL, D), jnp.float32),
    )(x0, sqdeg, y1, y2, y3)


def _pad_edges(idx, pad_val):
    out = jnp.full((EPAD,), pad_val, jnp.int32)
    out = lax.dynamic_update_slice(out, idx.astype(jnp.int32), (0,))
    return out.reshape(TILES, NCH, CHUNK)


@jax.jit
def kernel(user_emb, item_emb, edge_index):
    u = edge_index[0]
    i = edge_index[1]

    ed = jnp.stack([
        jnp.stack([_pad_edges(i + PAD_HALF, PAD_ROW), _pad_edges(u, PAD_ROW)], axis=2),
        jnp.stack([_pad_edges(u, PAD_ROW), _pad_edges(i, PAD_ROW)], axis=2),
    ])

    x0 = jnp.zeros((NFULL, D), jnp.float32)
    x0 = lax.dynamic_update_slice(x0, user_emb, (0, 0))
    x0 = lax.dynamic_update_slice(x0, item_emb, (PAD_HALF, 0))

    deg = _sc_degree(ed)
    y0, dinv2, sqdeg = _tc_scale0(deg.reshape(NFULL, 1), x0)
    d2 = dinv2.reshape(NFULL)

    y1 = _sc_spmm(ed, y0, d2)
    y2 = _sc_spmm(ed, y1, d2)
    y3 = _sc_spmm(ed, y2, d2)

    s = _tc_final(x0, sqdeg, y1, y2, y3)
    return s[:NU], s[PAD_HALF:PAD_HALF + NU]


# speedup vs baseline: 20.3782x; 1.1087x over previous
import functools

import jax
import jax.numpy as jnp
from jax import lax
from jax.experimental import pallas as pl
from jax.experimental.pallas import tpu as pltpu
from jax.experimental.pallas import tpu_sc as plsc

NU = 25000
D = 64
E = 400000
TILES = 16
LANES = 16
CHUNK = 128
PAD_HALF = 25088
NFULL = 2 * PAD_HALF
ROWS_PT = PAD_HALF // TILES
EPAD = 401408
NCH = EPAD // (TILES * CHUNK)
PAD_ROW = NU

_mesh = plsc.VectorSubcoreMesh(core_axis_name="c", subcore_axis_name="s")


def _zero_buf(rows_v):
    def body(r, _):
        for j in range(D // LANES):
            rows_v[r, pl.ds(j * LANES, LANES)] = jnp.zeros((LANES,), jnp.float32)
        return 0
    lax.fori_loop(0, CHUNK, body, 0)


@functools.partial(
    pl.kernel,
    out_type=jax.ShapeDtypeStruct((NFULL,), jnp.float32),
    mesh=_mesh,
    scratch_types=[
        pltpu.VMEM((NCH, 2, CHUNK), jnp.int32),
        pltpu.VMEM((CHUNK,), jnp.float32),
        pltpu.VMEM((ROWS_PT,), jnp.float32),
        pltpu.VMEM_SHARED((PAD_HALF,), jnp.float32),
        pltpu.SemaphoreType.DMA,
        pltpu.SemaphoreType.DMA,
        pltpu.SemaphoreType.DMA,
        pltpu.SemaphoreType.DMA,
    ],
)
def _sc_degree(dst_hbm, deg_hbm, idx_all, ones_v, zrow_v, deg_sh, s0, s1, s2, s3):
    c = lax.axis_index("c")
    s = lax.axis_index("s")
    sems = (s0, s1, s2, s3)

    pltpu.sync_copy(dst_hbm.at[c, s], idx_all)

    def fill(r, _):
        ones_v[pl.ds(r * LANES, LANES)] = jnp.ones((LANES,), jnp.float32)
        return 0
    lax.fori_loop(0, CHUNK // LANES, fill, 0)

    def zfill(r, _):
        zrow_v[pl.ds(r * LANES, LANES)] = jnp.zeros((LANES,), jnp.float32)
        return 0
    lax.fori_loop(0, ROWS_PT // LANES, zfill, 0)
    pltpu.sync_copy(zrow_v, deg_sh.at[pl.ds(s * ROWS_PT, ROWS_PT)])
    plsc.subcore_barrier()

    for b in range(4):
        pltpu.async_copy(ones_v, deg_sh.at[idx_all.at[b, 1]], sems[b], add=True)

    def edge(j, _):
        for b in range(4):
            k = 4 * j + b
            pltpu.make_async_copy(
                ones_v, deg_sh.at[idx_all.at[k, 1]], sems[b]).wait()

            @pl.when(k + 4 < NCH)
            def _():
                pltpu.async_copy(
                    ones_v, deg_sh.at[idx_all.at[k + 4, 1]], sems[b], add=True)
        return 0
    lax.fori_loop(0, NCH // 4, edge, 0)
    plsc.subcore_barrier()

    pltpu.sync_copy(deg_sh.at[pl.ds(s * ROWS_PT, ROWS_PT)], zrow_v)
    pltpu.sync_copy(zrow_v, deg_hbm.at[pl.ds(c * PAD_HALF + s * ROWS_PT, ROWS_PT)])


@functools.partial(
    pl.kernel,
    out_type=jax.ShapeDtypeStruct((NFULL, D), jnp.float32),
    mesh=_mesh,
    scratch_types=[
        pltpu.VMEM((3, 2, CHUNK), jnp.int32),
        pltpu.VMEM((3, CHUNK, D), jnp.float32),
        pltpu.VMEM((CHUNK,), jnp.float32),
        pltpu.VMEM_SHARED((PAD_HALF, D), jnp.float32),
        pltpu.SemaphoreType.DMA,
        pltpu.SemaphoreType.DMA,
        pltpu.SemaphoreType.DMA,
    ],
    compiler_params=pltpu.CompilerParams(
        use_tc_tiling_on_sc=False, needs_layout_passes=False),
)
def _sc_spmm(ed_hbm, y_hbm, dinv2_hbm, out_hbm, id3, rows3, dv_v,
             acc_sh, s0, s1, s2):
    c = lax.axis_index("c")
    s = lax.axis_index("s")
    base = s * ROWS_PT
    sems = (s0, s1, s2)
    rows_a = rows3.at[0]

    _zero_buf(rows_a)
    def zcp(k, _):
        pltpu.sync_copy(rows_a, acc_sh.at[pl.ds(base + k * CHUNK, CHUNK)])
        return 0
    lax.fori_loop(0, ROWS_PT // CHUNK, zcp, 0)
    rem = ROWS_PT - (ROWS_PT // CHUNK) * CHUNK
    if rem:
        pltpu.sync_copy(
            rows_a.at[pl.ds(0, rem)],
            acc_sh.at[pl.ds(base + (ROWS_PT // CHUNK) * CHUNK, rem)],
        )
    plsc.subcore_barrier()

    for b in range(3):
        pltpu.sync_copy(ed_hbm.at[c, s, b], id3.at[b])
        pltpu.async_copy(y_hbm.at[id3.at[b, 0]], rows3.at[b], sems[b])

    def edge(j, _):
        for b in range(3):
            k = 3 * j + b
            pltpu.make_async_copy(
                y_hbm.at[id3.at[b, 0]], rows3.at[b], sems[b]).wait()
            pltpu.sync_copy(rows3.at[b], acc_sh.at[id3.at[b, 1]], add=True)

            @pl.when(k + 3 < NCH)
            def _():
                pltpu.sync_copy(ed_hbm.at[c, s, k + 3], id3.at[b])
                pltpu.async_copy(y_hbm.at[id3.at[b, 0]], rows3.at[b], sems[b])
        return 0
    lax.fori_loop(0, (NCH - 1) // 3, edge, 0)

    pltpu.make_async_copy(y_hbm.at[id3.at[0, 0]], rows3.at[0], sems[0]).wait()
    pltpu.sync_copy(rows3.at[0], acc_sh.at[id3.at[0, 1]], add=True)
    plsc.subcore_barrier()

    gbase = c * PAD_HALF + base

    def scale_rows(nrows):
        def srow(r, _):
            dv = plsc.load_gather(dv_v, [jnp.full((LANES,), r, jnp.int32)])
            for j in range(D // LANES):
                sl = pl.ds(j * LANES, LANES)
                rows_a[r, sl] = dv * rows_a[r, sl]
            return 0
        lax.fori_loop(0, nrows, srow, 0)

    def wout(k, _):
        pltpu.sync_copy(acc_sh.at[pl.ds(base + k * CHUNK, CHUNK)], rows_a)
        pltpu.sync_copy(dinv2_hbm.at[pl.ds(gbase + k * CHUNK, CHUNK)], dv_v)
        scale_rows(CHUNK)
        pltpu.sync_copy(rows_a, out_hbm.at[pl.ds(gbase + k * CHUNK, CHUNK)])
        return 0
    lax.fori_loop(0, ROWS_PT // CHUNK, wout, 0)
    if rem:
        tail = (ROWS_PT // CHUNK) * CHUNK
        pltpu.sync_copy(acc_sh.at[pl.ds(base + tail, rem)], rows_a.at[pl.ds(0, rem)])
        pltpu.sync_copy(dinv2_hbm.at[pl.ds(gbase + tail, rem)], dv_v.at[pl.ds(0, rem)])
        scale_rows(rem)
        pltpu.sync_copy(rows_a.at[pl.ds(0, rem)], out_hbm.at[pl.ds(gbase + tail, rem)])


_BLK = 448


def _tc_scale0_body(deg_ref, x_ref, y_ref, dinv2_ref, sqdeg_ref):
    d = deg_ref[...]
    pos = d > 0
    dsafe = jnp.where(pos, d, 1.0)
    dinv = jnp.where(pos, lax.rsqrt(dsafe), 0.0)
    dinv2_ref[...] = jnp.where(pos, 1.0 / dsafe, 0.0)
    sqdeg_ref[...] = jnp.sqrt(d)
    y_ref[...] = dinv * x_ref[...]


def _tc_final_body(x0_ref, sq_ref, y1_ref, y2_ref, y3_ref, s_ref):
    ysum = y1_ref[...] + y2_ref[...] + y3_ref[...]
    s_ref[...] = (x0_ref[...] + sq_ref[...] * ysum) * 0.25


def _rows(block_cols):
    return pl.BlockSpec((_BLK, block_cols), lambda i: (i, 0))


def _tc_scale0(deg2, x0):
    return pl.pallas_call(
        _tc_scale0_body,
        grid=(NFULL // _BLK,),
        in_specs=[_rows(1), _rows(D)],
        out_specs=[_rows(D), _rows(1), _rows(1)],
        out_shape=[
            jax.ShapeDtypeStruct((NFULL, D), jnp.float32),
            jax.ShapeDtypeStruct((NFULL, 1), jnp.float32),
            jax.ShapeDtypeStruct((NFULL, 1), jnp.float32),
        ],
    )(deg2, x0)


def _tc_final(x0, sqdeg, y1, y2, y3):
    return pl.pallas_call(
        _tc_final_body,
        grid=(NFULL // _BLK,),
        in_specs=[_rows(D), _rows(1), _rows(D), _rows(D), _rows(D)],
        out_specs=_rows(D),
        out_shape=jax.ShapeDtypeStruct((NFULL, D), jnp.float32),
    )(x0, sqdeg, y1, y2, y3)


def _pad_edges(idx, pad_val):
    out = jnp.full((EPAD,), pad_val, jnp.int32)
    out = lax.dynamic_update_slice(out, idx.astype(jnp.int32), (0,))
    return out.reshape(TILES, NCH, CHUNK)


@jax.jit
def kernel(user_emb, item_emb, edge_index):
    u = edge_index[0]
    i = edge_index[1]

    ed = jnp.stack([
        jnp.stack([_pad_edges(i + PAD_HALF, PAD_ROW), _pad_edges(u, PAD_ROW)], axis=2),
        jnp.stack([_pad_edges(u, PAD_ROW), _pad_edges(i, PAD_ROW)], axis=2),
    ])

    x0 = jnp.zeros((NFULL, D), jnp.float32)
    x0 = lax.dynamic_update_slice(x0, user_emb, (0, 0))
    x0 = lax.dynamic_update_slice(x0, item_emb, (PAD_HALF, 0))

    deg = _sc_degree(ed)
    y0, dinv2, sqdeg = _tc_scale0(deg.reshape(NFULL, 1), x0)
    d2 = dinv2.reshape(NFULL)

    y1 = _sc_spmm(ed, y0, d2)
    y2 = _sc_spmm(ed, y1, d2)
    y3 = _sc_spmm(ed, y2, d2)

    s = _tc_final(x0, sqdeg, y1, y2, y3)
    return s[:NU], s[PAD_HALF:PAD_HALF + NU]


# speedup vs baseline: 21.6406x; 1.0619x over previous
import functools

import jax
import jax.numpy as jnp
from jax import lax
from jax.experimental import pallas as pl
from jax.experimental.pallas import tpu as pltpu
from jax.experimental.pallas import tpu_sc as plsc

NU = 25000
D = 64
E = 400000
TILES = 16
LANES = 16
CHUNK = 128
PAD_HALF = 25088
NFULL = 2 * PAD_HALF
ROWS_PT = PAD_HALF // TILES
EPAD = 401408
NCH = EPAD // (TILES * CHUNK)
PAD_ROW = NU

_mesh = plsc.VectorSubcoreMesh(core_axis_name="c", subcore_axis_name="s")


def _zero_buf(rows_v):
    def body(r, _):
        for j in range(D // LANES):
            rows_v[r, pl.ds(j * LANES, LANES)] = jnp.zeros((LANES,), jnp.float32)
        return 0
    lax.fori_loop(0, CHUNK, body, 0)


def _rsqrt16(d):
    i = plsc.bitcast(d, jnp.int32)
    i = jnp.int32(0x5F3759DF) - lax.shift_right_logical(i, 1)
    y = plsc.bitcast(i, jnp.float32)
    for _ in range(3):
        y = y * (1.5 - 0.5 * d * y * y)
    return jnp.where(d > 0, y, 0.0)


@functools.partial(
    pl.kernel,
    out_type=[
        jax.ShapeDtypeStruct((NFULL, D), jnp.float32),
        jax.ShapeDtypeStruct((NFULL,), jnp.float32),
        jax.ShapeDtypeStruct((NFULL,), jnp.float32),
    ],
    mesh=_mesh,
    scratch_types=[
        pltpu.VMEM((NCH, 2, CHUNK), jnp.int32),
        pltpu.VMEM((CHUNK,), jnp.float32),
        pltpu.VMEM((ROWS_PT,), jnp.float32),
        pltpu.VMEM((CHUNK, D), jnp.float32),
        pltpu.VMEM((CHUNK,), jnp.float32),
        pltpu.VMEM((CHUNK,), jnp.float32),
        pltpu.VMEM((CHUNK,), jnp.float32),
        pltpu.VMEM_SHARED((PAD_HALF,), jnp.float32),
        pltpu.SemaphoreType.DMA,
        pltpu.SemaphoreType.DMA,
        pltpu.SemaphoreType.DMA,
        pltpu.SemaphoreType.DMA,
    ],
    compiler_params=pltpu.CompilerParams(
        use_tc_tiling_on_sc=False, needs_layout_passes=False),
)
def _sc_degree(dst_hbm, x0_hbm, y0_hbm, dinv2_hbm, sqdeg_hbm,
               idx_all, ones_v, zrow_v, xrows_v, dv_v, d2_v, sq_v,
               deg_sh, s0, s1, s2, s3):
    c = lax.axis_index("c")
    s = lax.axis_index("s")
    sems = (s0, s1, s2, s3)

    pltpu.sync_copy(dst_hbm.at[c, s], idx_all)

    def fill(r, _):
        ones_v[pl.ds(r * LANES, LANES)] = jnp.ones((LANES,), jnp.float32)
        return 0
    lax.fori_loop(0, CHUNK // LANES, fill, 0)

    def zfill(r, _):
        zrow_v[pl.ds(r * LANES, LANES)] = jnp.zeros((LANES,), jnp.float32)
        return 0
    lax.fori_loop(0, ROWS_PT // LANES, zfill, 0)
    pltpu.sync_copy(zrow_v, deg_sh.at[pl.ds(s * ROWS_PT, ROWS_PT)])
    plsc.subcore_barrier()

    for b in range(4):
        pltpu.async_copy(ones_v, deg_sh.at[idx_all.at[b, 1]], sems[b], add=True)

    def edge(j, _):
        for b in range(4):
            k = 4 * j + b
            pltpu.make_async_copy(
                ones_v, deg_sh.at[idx_all.at[k, 1]], sems[b]).wait()

            @pl.when(k + 4 < NCH)
            def _():
                pltpu.async_copy(
                    ones_v, deg_sh.at[idx_all.at[k + 4, 1]], sems[b], add=True)
        return 0
    lax.fori_loop(0, NCH // 4, edge, 0)
    plsc.subcore_barrier()

    base = s * ROWS_PT
    gbase = c * PAD_HALF + base

    def node_chunk(off, nrows):
        pltpu.sync_copy(deg_sh.at[pl.ds(base + off, nrows)],
                        dv_v.at[pl.ds(0, nrows)])

        def vec(r, _):
            sl = pl.ds(r * LANES, LANES)
            d = dv_v[sl]
            dinv = _rsqrt16(d)
            dv_v[sl] = dinv
            d2_v[sl] = dinv * dinv
            sq_v[sl] = d * dinv
            return 0
        lax.fori_loop(0, nrows // LANES, vec, 0)
        pltpu.sync_copy(d2_v.at[pl.ds(0, nrows)],
                        dinv2_hbm.at[pl.ds(gbase + off, nrows)])
        pltpu.sync_copy(sq_v.at[pl.ds(0, nrows)],
                        sqdeg_hbm.at[pl.ds(gbase + off, nrows)])

        pltpu.sync_copy(x0_hbm.at[pl.ds(gbase + off, nrows)],
                        xrows_v.at[pl.ds(0, nrows)])

        def srow(r, _):
            dv = plsc.load_gather(dv_v, [jnp.full((LANES,), r, jnp.int32)])
            for j in range(D // LANES):
                sl = pl.ds(j * LANES, LANES)
                xrows_v[r, sl] = dv * xrows_v[r, sl]
            return 0
        lax.fori_loop(0, nrows, srow, 0)
        pltpu.sync_copy(xrows_v.at[pl.ds(0, nrows)],
                        y0_hbm.at[pl.ds(gbase + off, nrows)])

    def wchunk(k, _):
        node_chunk(k * CHUNK, CHUNK)
        return 0
    lax.fori_loop(0, ROWS_PT // CHUNK, wchunk, 0)
    node_chunk((ROWS_PT // CHUNK) * CHUNK, ROWS_PT - (ROWS_PT // CHUNK) * CHUNK)


@functools.partial(
    pl.kernel,
    out_type=jax.ShapeDtypeStruct((NFULL, D), jnp.float32),
    mesh=_mesh,
    scratch_types=[
        pltpu.VMEM((3, 2, CHUNK), jnp.int32),
        pltpu.VMEM((3, CHUNK, D), jnp.float32),
        pltpu.VMEM((CHUNK,), jnp.float32),
        pltpu.VMEM_SHARED((PAD_HALF, D), jnp.float32),
        pltpu.SemaphoreType.DMA,
        pltpu.SemaphoreType.DMA,
        pltpu.SemaphoreType.DMA,
    ],
    compiler_params=pltpu.CompilerParams(
        use_tc_tiling_on_sc=False, needs_layout_passes=False),
)
def _sc_spmm(ed_hbm, y_hbm, dinv2_hbm, out_hbm, id3, rows3, dv_v,
             acc_sh, s0, s1, s2):
    c = lax.axis_index("c")
    s = lax.axis_index("s")
    base = s * ROWS_PT
    sems = (s0, s1, s2)
    rows_a = rows3.at[0]

    _zero_buf(rows_a)
    def zcp(k, _):
        pltpu.sync_copy(rows_a, acc_sh.at[pl.ds(base + k * CHUNK, CHUNK)])
        return 0
    lax.fori_loop(0, ROWS_PT // CHUNK, zcp, 0)
    rem = ROWS_PT - (ROWS_PT // CHUNK) * CHUNK
    if rem:
        pltpu.sync_copy(
            rows_a.at[pl.ds(0, rem)],
            acc_sh.at[pl.ds(base + (ROWS_PT // CHUNK) * CHUNK, rem)],
        )
    plsc.subcore_barrier()

    for b in range(3):
        pltpu.sync_copy(ed_hbm.at[c, s, b], id3.at[b])
        pltpu.async_copy(y_hbm.at[id3.at[b, 0]], rows3.at[b], sems[b])

    def edge(j, _):
        for b in range(3):
            k = 3 * j + b
            pltpu.make_async_copy(
                y_hbm.at[id3.at[b, 0]], rows3.at[b], sems[b]).wait()
            pltpu.sync_copy(rows3.at[b], acc_sh.at[id3.at[b, 1]], add=True)

            @pl.when(k + 3 < NCH)
            def _():
                pltpu.sync_copy(ed_hbm.at[c, s, k + 3], id3.at[b])
                pltpu.async_copy(y_hbm.at[id3.at[b, 0]], rows3.at[b], sems[b])
        return 0
    lax.fori_loop(0, (NCH - 1) // 3, edge, 0)

    pltpu.make_async_copy(y_hbm.at[id3.at[0, 0]], rows3.at[0], sems[0]).wait()
    pltpu.sync_copy(rows3.at[0], acc_sh.at[id3.at[0, 1]], add=True)
    plsc.subcore_barrier()

    gbase = c * PAD_HALF + base

    def scale_rows(nrows):
        def srow(r, _):
            dv = plsc.load_gather(dv_v, [jnp.full((LANES,), r, jnp.int32)])
            for j in range(D // LANES):
                sl = pl.ds(j * LANES, LANES)
                rows_a[r, sl] = dv * rows_a[r, sl]
            return 0
        lax.fori_loop(0, nrows, srow, 0)

    def wout(k, _):
        pltpu.sync_copy(acc_sh.at[pl.ds(base + k * CHUNK, CHUNK)], rows_a)
        pltpu.sync_copy(dinv2_hbm.at[pl.ds(gbase + k * CHUNK, CHUNK)], dv_v)
        scale_rows(CHUNK)
        pltpu.sync_copy(rows_a, out_hbm.at[pl.ds(gbase + k * CHUNK, CHUNK)])
        return 0
    lax.fori_loop(0, ROWS_PT // CHUNK, wout, 0)
    if rem:
        tail = (ROWS_PT // CHUNK) * CHUNK
        pltpu.sync_copy(acc_sh.at[pl.ds(base + tail, rem)], rows_a.at[pl.ds(0, rem)])
        pltpu.sync_copy(dinv2_hbm.at[pl.ds(gbase + tail, rem)], dv_v.at[pl.ds(0, rem)])
        scale_rows(rem)
        pltpu.sync_copy(rows_a.at[pl.ds(0, rem)], out_hbm.at[pl.ds(gbase + tail, rem)])


_BLK = 448


def _tc_final_body(x0_ref, sq_ref, y1_ref, y2_ref, y3_ref, s_ref):
    ysum = y1_ref[...] + y2_ref[...] + y3_ref[...]
    s_ref[...] = (x0_ref[...] + sq_ref[...] * ysum) * 0.25


def _rows(block_cols):
    return pl.BlockSpec((_BLK, block_cols), lambda i: (i, 0))


def _tc_final(x0, sqdeg, y1, y2, y3):
    return pl.pallas_call(
        _tc_final_body,
        grid=(NFULL // _BLK,),
        in_specs=[_rows(D), _rows(1), _rows(D), _rows(D), _rows(D)],
        out_specs=_rows(D),
        out_shape=jax.ShapeDtypeStruct((NFULL, D), jnp.float32),
    )(x0, sqdeg, y1, y2, y3)


def _pad_edges(idx, pad_val):
    out = jnp.full((EPAD,), pad_val, jnp.int32)
    out = lax.dynamic_update_slice(out, idx.astype(jnp.int32), (0,))
    return out.reshape(TILES, NCH, CHUNK)


@jax.jit
def kernel(user_emb, item_emb, edge_index):
    u = edge_index[0]
    i = edge_index[1]

    ed = jnp.stack([
        jnp.stack([_pad_edges(i + PAD_HALF, PAD_ROW), _pad_edges(u, PAD_ROW)], axis=2),
        jnp.stack([_pad_edges(u, PAD_ROW), _pad_edges(i, PAD_ROW)], axis=2),
    ])

    x0 = jnp.zeros((NFULL, D), jnp.float32)
    x0 = lax.dynamic_update_slice(x0, user_emb, (0, 0))
    x0 = lax.dynamic_update_slice(x0, item_emb, (PAD_HALF, 0))

    y0, d2, sqdeg = _sc_degree(ed, x0)

    y1 = _sc_spmm(ed, y0, d2)
    y2 = _sc_spmm(ed, y1, d2)
    y3 = _sc_spmm(ed, y2, d2)

    s = _tc_final(x0, sqdeg.reshape(NFULL, 1), y1, y2, y3)
    return s[:NU], s[PAD_HALF:PAD_HALF + NU]
